# baseline (speedup 1.0000x reference)
import functools

import jax
import jax.numpy as jnp
from jax import lax
from jax.experimental import pallas as pl
from jax.experimental.pallas import tpu as pltpu
from jax.experimental.pallas import tpu_sc as plsc

T = 2048
H = 1024
I = 512
E = 8
K = 2
B = 256
NUM_BLOCKS = 23
R_PAD = NUM_BLOCKS * B
NC, NS = 2, 16
NW = NC * NS
TOKS_PER_W = T // NW

_SC_MESH = plsc.VectorSubcoreMesh(core_axis_name="c", subcore_axis_name="s")


def _worker_id():
    return lax.axis_index("s") * NC + lax.axis_index("c")


@functools.partial(
    pl.kernel,
    out_type=(
        jax.ShapeDtypeStruct((R_PAD, H), jnp.float32),
        jax.ShapeDtypeStruct((R_PAD, 128), jnp.float32),
    ),
    mesh=_SC_MESH,
    name="sc_dispatch_scatter",
    scratch_types=[
        pltpu.VMEM((K, TOKS_PER_W), jnp.int32),
        pltpu.VMEM((TOKS_PER_W, H), jnp.float32),
        pltpu.VMEM((K, TOKS_PER_W, 128), jnp.float32),
        pltpu.SemaphoreType.DMA,
        pltpu.SemaphoreType.DMA,
        pltpu.SemaphoreType.DMA,
        pltpu.SemaphoreType.DMA,
    ],
)
def _dispatch(hid_hbm, pos3_hbm, w16_hbm, xs_hbm, ws_hbm,
              idx2_v, rows_v, wbuf_v, s0, s1, s2, s3):
    wid = _worker_id()
    base = wid * TOKS_PER_W
    pltpu.sync_copy(pos3_hbm.at[wid], idx2_v)
    pltpu.sync_copy(w16_hbm.at[wid], wbuf_v)
    pltpu.sync_copy(hid_hbm.at[pl.ds(base, TOKS_PER_W)], rows_v)
    ce = pltpu.async_copy(rows_v, xs_hbm.at[idx2_v.at[0]], s0)
    co = pltpu.async_copy(rows_v, xs_hbm.at[idx2_v.at[1]], s1)
    cwe = pltpu.async_copy(wbuf_v.at[0], ws_hbm.at[idx2_v.at[0]], s2)
    cwo = pltpu.async_copy(wbuf_v.at[1], ws_hbm.at[idx2_v.at[1]], s3)
    ce.wait()
    co.wait()
    cwe.wait()
    cwo.wait()


def _ffn_body(ge_ref, x_ref, gu_ref, dp_ref, w_ref, o_ref):
    del ge_ref
    x = x_ref[...]
    w1 = gu_ref[0]
    xw = lax.dot_general(x, w1, (((1,), (1,)), ((), ())),
                         preferred_element_type=jnp.float32)
    gate = xw[:, :I]
    up = xw[:, I:]
    h = gate * lax.logistic(gate) * up
    w2 = dp_ref[0]
    out = lax.dot_general(h, w2, (((1,), (1,)), ((), ())),
                          preferred_element_type=jnp.float32)
    o_ref[...] = out * w_ref[:, 0:1]


_ffn = pl.pallas_call(
    _ffn_body,
    grid_spec=pltpu.PrefetchScalarGridSpec(
        num_scalar_prefetch=1,
        grid=(NUM_BLOCKS,),
        in_specs=[
            pl.BlockSpec((B, H), lambda b, ge: (b, 0)),
            pl.BlockSpec((1, 2 * I, H), lambda b, ge: (ge[b], 0, 0)),
            pl.BlockSpec((1, H, I), lambda b, ge: (ge[b], 0, 0)),
            pl.BlockSpec((B, 128), lambda b, ge: (b, 0)),
        ],
        out_specs=pl.BlockSpec((B, H), lambda b, ge: (b, 0)),
    ),
    out_shape=jax.ShapeDtypeStruct((R_PAD, H), jnp.float32),
)


_CR = 16
_CN = TOKS_PER_W // _CR


@functools.partial(
    pl.kernel,
    out_type=jax.ShapeDtypeStruct((T, H), jnp.float32),
    mesh=_SC_MESH,
    name="sc_combine_gather",
    scratch_types=[
        pltpu.VMEM((TOKS_PER_W,), jnp.int32),
        pltpu.VMEM((TOKS_PER_W,), jnp.int32),
        pltpu.VMEM((_CR, H), jnp.float32),
        pltpu.VMEM((_CR, H), jnp.float32),
        pltpu.VMEM((_CR, H), jnp.float32),
        pltpu.VMEM((_CR, H), jnp.float32),
        pltpu.VMEM((_CR, H), jnp.float32),
        pltpu.VMEM((_CR, H), jnp.float32),
        pltpu.SemaphoreType.DMA,
        pltpu.SemaphoreType.DMA,
        pltpu.SemaphoreType.DMA,
        pltpu.SemaphoreType.DMA,
        pltpu.SemaphoreType.DMA,
        pltpu.SemaphoreType.DMA,
        pltpu.SemaphoreType.DMA,
        pltpu.SemaphoreType.DMA,
        pltpu.SemaphoreType.DMA,
    ],
)
def _combine_gather(os_hbm, pe_hbm, po_hbm, out_hbm,
                    ie_v, io_v, be0, bo0, be1, bo1, be2, bo2,
                    ge0, go0, ge1, go1, ge2, go2, ss0, ss1, ss2):
    base = _worker_id() * TOKS_PER_W
    pltpu.sync_copy(pe_hbm.at[pl.ds(base, TOKS_PER_W)], ie_v)
    pltpu.sync_copy(po_hbm.at[pl.ds(base, TOKS_PER_W)], io_v)
    bes = (be0, be1, be2)
    bos = (bo0, bo1, bo2)
    gesem = (ge0, ge1, ge2)
    gosem = (go0, go1, go2)
    ssem = (ss0, ss1, ss2)
    nbuf = 3

    def _gather(r):
        b = r % nbuf
        ce = pltpu.async_copy(
            os_hbm.at[ie_v.at[pl.ds(r * _CR, _CR)]], bes[b], gesem[b])
        co = pltpu.async_copy(
            os_hbm.at[io_v.at[pl.ds(r * _CR, _CR)]], bos[b], gosem[b])
        return ce, co

    gathers = [None] * _CN
    scatters = [None] * _CN
    for r in range(min(nbuf - 1, _CN)):
        gathers[r] = _gather(r)
    for r in range(_CN):
        b = r % nbuf
        pre = r + nbuf - 1
        if pre < _CN:
            if scatters[pre - nbuf] is not None:
                scatters[pre - nbuf].wait()
            gathers[pre] = _gather(pre)
        gathers[r][0].wait()
        gathers[r][1].wait()
        be_v = bes[b]
        bo_v = bos[b]

        def _add_row(i, _):
            for s in range(H // 16):
                be_v[i, pl.ds(s * 16, 16)] = (
                    be_v[i, pl.ds(s * 16, 16)] + bo_v[i, pl.ds(s * 16, 16)]
                )
            return 0

        lax.fori_loop(0, _CR, _add_row, 0)
        scatters[r] = pltpu.async_copy(
            bes[b], out_hbm.at[pl.ds(base + r * _CR, _CR)], ssem[b])
    for r in range(max(0, _CN - nbuf), _CN):
        scatters[r].wait()


def kernel(hidden_states, top_k_indices, top_k_weights, gate_up_proj, down_proj):
    e_flat = top_k_indices.reshape(-1).astype(jnp.int32)
    w_flat = top_k_weights.reshape(-1)
    onehot = (e_flat[:, None] == jnp.arange(E, dtype=jnp.int32)[None, :]
              ).astype(jnp.int32)
    ranks_inc = jnp.cumsum(onehot, axis=0)
    counts = ranks_inc[-1]
    rank = jnp.sum(ranks_inc * onehot, axis=1) - 1
    padded = ((counts + B - 1) // B) * B
    pend = jnp.cumsum(padded)
    pstart = pend - padded
    pos_flat = (pstart[e_flat] + rank).astype(jnp.int32)
    block_expert = jnp.minimum(
        jnp.searchsorted(pend, jnp.arange(NUM_BLOCKS, dtype=jnp.int32) * B,
                         side="right"),
        E - 1,
    ).astype(jnp.int32)
    pos_tk = pos_flat.reshape(T, K)
    pos_e = pos_tk[:, 0]
    pos_o = pos_tk[:, 1]
    pos3 = jnp.stack([pos_e.reshape(NW, TOKS_PER_W),
                      pos_o.reshape(NW, TOKS_PER_W)], axis=1)
    w3 = w_flat.reshape(NW, TOKS_PER_W, K).transpose(0, 2, 1)
    w16 = jnp.broadcast_to(w3[..., None], (NW, K, TOKS_PER_W, 128))

    x_sorted, w_rows = _dispatch(hidden_states, pos3, w16)
    out_sorted = _ffn(block_expert, x_sorted, gate_up_proj, down_proj,
                      w_rows)
    return _combine_gather(out_sorted, pos_e, pos_o)
